# Initial kernel scaffold; baseline (speedup 1.0000x reference)
import jax
import jax.numpy as jnp
from jax.experimental import pallas as pl


def kernel(embedding, gene_with_go_idx, gene_with_go_value, gene_without_go_value, other_info, W1, b1, W2, b2, W3, b3):
    raise NotImplementedError("write your pallas kernel here")


# speedup vs baseline: 1.0517x; 1.0517x over previous
import functools

import jax
import jax.numpy as jnp
from jax import lax
from jax.experimental import pallas as pl
from jax.experimental.pallas import tpu as pltpu
from jax.experimental.pallas import tpu_sc as plsc

B, S, D, V = 4096, 200, 64, 1000000
BS = B * S
OTHER = 64


_ROWS_BLK = 8000


def _rowsum_body(emb_ref, out_ref):
    out_ref[...] = jnp.sum(emb_ref[...], axis=1).reshape(1, 1, _ROWS_BLK)


_rowsum_call = pl.pallas_call(
    _rowsum_body,
    grid=(V // _ROWS_BLK,),
    in_specs=[pl.BlockSpec((_ROWS_BLK, D), lambda i: (i, 0))],
    out_specs=pl.BlockSpec((1, 1, _ROWS_BLK), lambda i: (i, 0, 0)),
    out_shape=jax.ShapeDtypeStruct((V // _ROWS_BLK, 1, _ROWS_BLK), jnp.float32),
)


_NC, _NS = 2, 16
_NW = _NC * _NS
_N_PER = BS // _NW


def _sc_gather_body(sums_hbm, idx_hbm, out_hbm, idx_v, vals_v, sem):
    wid = lax.axis_index("s") * _NC + lax.axis_index("c")
    base = wid * _N_PER
    pltpu.sync_copy(idx_hbm.at[pl.ds(base, _N_PER)], idx_v)
    pltpu.async_copy(sums_hbm.at[idx_v], vals_v, sem).wait()
    pltpu.sync_copy(vals_v, out_hbm.at[pl.ds(base, _N_PER)])


@functools.cache
def _sc_gather_call():
    return pl.kernel(
        _sc_gather_body,
        out_type=jax.ShapeDtypeStruct((BS,), jnp.float32),
        mesh=plsc.VectorSubcoreMesh(
            core_axis_name="c", subcore_axis_name="s", num_cores=_NC,
            num_subcores=_NS),
        scratch_types=[
            pltpu.VMEM((_N_PER,), jnp.int32),
            pltpu.VMEM((_N_PER,), jnp.float32),
            pltpu.SemaphoreType.DMA,
        ],
    )


_B_BLK = 512


def _mlp_body(g_ref, gwv_ref, gwov_ref, oth_ref, w1a_ref, w1b_ref, w1c_ref,
              w1d_ref, b1_ref, w2_ref, b2_ref, w3_ref, b3_ref, out_ref):
    gwv = gwv_ref[...]
    emb = g_ref[...] * gwv * (1.0 / D)
    h = (
        jnp.dot(emb, w1a_ref[...], preferred_element_type=jnp.float32)
        + jnp.dot(gwv, w1b_ref[...], preferred_element_type=jnp.float32)
        + jnp.dot(gwov_ref[...], w1c_ref[...], preferred_element_type=jnp.float32)
        + jnp.dot(oth_ref[...], w1d_ref[...], preferred_element_type=jnp.float32)
        + b1_ref[...]
    )
    h = jnp.maximum(h, 0.0)
    h = jnp.maximum(
        jnp.dot(h, w2_ref[...], preferred_element_type=jnp.float32) + b2_ref[...],
        0.0)
    out_ref[...] = (
        jnp.dot(h, w3_ref[...], preferred_element_type=jnp.float32) + b3_ref[...])


def _full(shape):
    return pl.BlockSpec(shape, lambda i: tuple(0 for _ in shape))


_mlp_call = pl.pallas_call(
    _mlp_body,
    grid=(B // _B_BLK,),
    in_specs=[
        pl.BlockSpec((_B_BLK, S), lambda i: (i, 0)),
        pl.BlockSpec((_B_BLK, S), lambda i: (i, 0)),
        pl.BlockSpec((_B_BLK, S), lambda i: (i, 0)),
        pl.BlockSpec((_B_BLK, OTHER), lambda i: (i, 0)),
        _full((S, 256)),
        _full((S, 256)),
        _full((S, 256)),
        _full((OTHER, 256)),
        _full((1, 256)),
        _full((256, 128)),
        _full((1, 128)),
        _full((128, 128)),
        _full((1, 128)),
    ],
    out_specs=pl.BlockSpec((_B_BLK, 128), lambda i: (i, 0)),
    out_shape=jax.ShapeDtypeStruct((B, 128), jnp.float32),
)


def kernel(embedding, gene_with_go_idx, gene_with_go_value,
           gene_without_go_value, other_info, W1, b1, W2, b2, W3, b3):
    sums = _rowsum_call(embedding).reshape(V)
    idx_flat = gene_with_go_idx.reshape(BS).astype(jnp.int32)
    gathered = _sc_gather_call()(sums, idx_flat)
    g2 = gathered.reshape(B, S)
    w1a = W1[0:S]
    w1b = W1[S:2 * S]
    w1c = W1[2 * S:3 * S]
    w1d = W1[3 * S:]
    return _mlp_call(
        g2, gene_with_go_value, gene_without_go_value, other_info,
        w1a, w1b, w1c, w1d, b1.reshape(1, 256),
        W2, b2.reshape(1, 128), W3, b3.reshape(1, 128))


# speedup vs baseline: 1.0563x; 1.0043x over previous
import functools

import jax
import jax.numpy as jnp
import numpy as np
from jax import lax
from jax.experimental import pallas as pl
from jax.experimental.pallas import tpu as pltpu
from jax.experimental.pallas import tpu_sc as plsc

B, S, D, V = 4096, 200, 64, 1000000
BS = B * S
OTHER = 64


_RPACK = 8
_FLATW = _RPACK * D
_VROWS = V // _RPACK
_ROWS_BLK = 1000


def _rowsum_body(emb_ref, ones_ref, out_ref):
    x = emb_ref[...].astype(jnp.bfloat16)
    out_ref[...] = jnp.dot(x, ones_ref[...], preferred_element_type=jnp.float32)


_rowsum_call = pl.pallas_call(
    _rowsum_body,
    grid=(_VROWS // _ROWS_BLK,),
    in_specs=[
        pl.BlockSpec((_ROWS_BLK, _FLATW), lambda i: (i, 0)),
        pl.BlockSpec((_FLATW, _RPACK), lambda i: (0, 0)),
    ],
    out_specs=pl.BlockSpec((_ROWS_BLK, _RPACK), lambda i: (i, 0)),
    out_shape=jax.ShapeDtypeStruct((_VROWS, _RPACK), jnp.float32),
)

_ONES_BLKDIAG = np.asarray(
    np.arange(_FLATW)[:, None] // D == np.arange(_RPACK)[None, :],
    dtype=np.float32).astype(jnp.bfloat16)


_NC, _NS = 2, 16
_NW = _NC * _NS
_N_PER = BS // _NW


def _sc_gather_body(sums_hbm, idx_hbm, out_hbm, idx_v, vals_v, sem):
    wid = lax.axis_index("s") * _NC + lax.axis_index("c")
    base = wid * _N_PER
    pltpu.sync_copy(idx_hbm.at[pl.ds(base, _N_PER)], idx_v)
    pltpu.async_copy(sums_hbm.at[idx_v], vals_v, sem).wait()
    pltpu.sync_copy(vals_v, out_hbm.at[pl.ds(base, _N_PER)])


@functools.cache
def _sc_gather_call():
    return pl.kernel(
        _sc_gather_body,
        out_type=jax.ShapeDtypeStruct((BS,), jnp.float32),
        mesh=plsc.VectorSubcoreMesh(
            core_axis_name="c", subcore_axis_name="s", num_cores=_NC,
            num_subcores=_NS),
        scratch_types=[
            pltpu.VMEM((_N_PER,), jnp.int32),
            pltpu.VMEM((_N_PER,), jnp.float32),
            pltpu.SemaphoreType.DMA,
        ],
    )


_B_BLK = 512


def _mlp_body(g_ref, gwv_ref, gwov_ref, oth_ref, w1a_ref, w1b_ref, w1c_ref,
              w1d_ref, b1_ref, w2_ref, b2_ref, w3_ref, b3_ref, out_ref):
    gwv = gwv_ref[...]
    emb = g_ref[...] * gwv * (1.0 / D)
    h = (
        jnp.dot(emb, w1a_ref[...], preferred_element_type=jnp.float32)
        + jnp.dot(gwv, w1b_ref[...], preferred_element_type=jnp.float32)
        + jnp.dot(gwov_ref[...], w1c_ref[...], preferred_element_type=jnp.float32)
        + jnp.dot(oth_ref[...], w1d_ref[...], preferred_element_type=jnp.float32)
        + b1_ref[...]
    )
    h = jnp.maximum(h, 0.0)
    h = jnp.maximum(
        jnp.dot(h, w2_ref[...], preferred_element_type=jnp.float32) + b2_ref[...],
        0.0)
    out_ref[...] = (
        jnp.dot(h, w3_ref[...], preferred_element_type=jnp.float32) + b3_ref[...])


def _full(shape):
    return pl.BlockSpec(shape, lambda i: tuple(0 for _ in shape))


_mlp_call = pl.pallas_call(
    _mlp_body,
    grid=(B // _B_BLK,),
    in_specs=[
        pl.BlockSpec((_B_BLK, S), lambda i: (i, 0)),
        pl.BlockSpec((_B_BLK, S), lambda i: (i, 0)),
        pl.BlockSpec((_B_BLK, S), lambda i: (i, 0)),
        pl.BlockSpec((_B_BLK, OTHER), lambda i: (i, 0)),
        _full((S, 256)),
        _full((S, 256)),
        _full((S, 256)),
        _full((OTHER, 256)),
        _full((1, 256)),
        _full((256, 128)),
        _full((1, 128)),
        _full((128, 128)),
        _full((1, 128)),
    ],
    out_specs=pl.BlockSpec((_B_BLK, 128), lambda i: (i, 0)),
    out_shape=jax.ShapeDtypeStruct((B, 128), jnp.float32),
)


def kernel(embedding, gene_with_go_idx, gene_with_go_value,
           gene_without_go_value, other_info, W1, b1, W2, b2, W3, b3):
    sums = _rowsum_call(embedding.reshape(_VROWS, _FLATW),
                        _ONES_BLKDIAG).reshape(V)
    idx_flat = gene_with_go_idx.reshape(BS).astype(jnp.int32)
    gathered = _sc_gather_call()(sums, idx_flat)
    g2 = gathered.reshape(B, S)
    w1a = W1[0:S]
    w1b = W1[S:2 * S]
    w1c = W1[2 * S:3 * S]
    w1d = W1[3 * S:]
    return _mlp_call(
        g2, gene_with_go_value, gene_without_go_value, other_info,
        w1a, w1b, w1c, w1d, b1.reshape(1, 256),
        W2, b2.reshape(1, 128), W3, b3.reshape(1, 128))


# speedup vs baseline: 5.0685x; 4.7983x over previous
import functools

import jax
import jax.numpy as jnp
import numpy as np
from jax import lax
from jax.experimental import pallas as pl
from jax.experimental.pallas import tpu as pltpu
from jax.experimental.pallas import tpu_sc as plsc

B, S, D, V = 4096, 200, 64, 1000000
BS = B * S
OTHER = 64


_COLS_BLK = 8192
_N_BLKS = -(-V // _COLS_BLK)
_VPAD = _N_BLKS * _COLS_BLK


def _rowsum_body(emb_ref, out_ref):
    out_ref[...] = jnp.sum(emb_ref[...], axis=0)


_rowsum_call = pl.pallas_call(
    _rowsum_body,
    grid=(_N_BLKS,),
    in_specs=[pl.BlockSpec((D, _COLS_BLK), lambda i: (0, i))],
    out_specs=pl.BlockSpec((_COLS_BLK,), lambda i: (i,)),
    out_shape=jax.ShapeDtypeStruct((_VPAD,), jnp.float32),
)


_NC, _NS = 2, 16
_NW = _NC * _NS
_N_PER = BS // _NW


def _sc_gather_body(sums_hbm, idx_hbm, out_hbm, idx_v, vals_v, sem):
    wid = lax.axis_index("s") * _NC + lax.axis_index("c")
    base = wid * _N_PER
    pltpu.sync_copy(idx_hbm.at[pl.ds(base, _N_PER)], idx_v)
    pltpu.async_copy(sums_hbm.at[idx_v], vals_v, sem).wait()
    pltpu.sync_copy(vals_v, out_hbm.at[pl.ds(base, _N_PER)])


@functools.cache
def _sc_gather_call():
    return pl.kernel(
        _sc_gather_body,
        out_type=jax.ShapeDtypeStruct((BS,), jnp.float32),
        mesh=plsc.VectorSubcoreMesh(
            core_axis_name="c", subcore_axis_name="s", num_cores=_NC,
            num_subcores=_NS),
        scratch_types=[
            pltpu.VMEM((_N_PER,), jnp.int32),
            pltpu.VMEM((_N_PER,), jnp.float32),
            pltpu.SemaphoreType.DMA,
        ],
    )


_B_BLK = 512


def _tdot(a_t, w):
    return lax.dot_general(a_t, w, (((0,), (0,)), ((), ())),
                           preferred_element_type=jnp.float32)


def _mlp_body(g_ref, gwv_ref, gwov_ref, oth_ref, w1a_ref, w1b_ref, w1c_ref,
              w1d_ref, b1_ref, w2_ref, b2_ref, w3_ref, b3_ref, out_ref):
    gwv_t = gwv_ref[...]
    emb_t = g_ref[...] * gwv_t * (1.0 / D)
    h = (_tdot(emb_t, w1a_ref[...]) + _tdot(gwv_t, w1b_ref[...])
         + _tdot(gwov_ref[...], w1c_ref[...]) + _tdot(oth_ref[...], w1d_ref[...])
         + b1_ref[...])
    h = jnp.maximum(h, 0.0)
    h = jnp.maximum(
        jnp.dot(h, w2_ref[...], preferred_element_type=jnp.float32) + b2_ref[...],
        0.0)
    out_ref[...] = (
        jnp.dot(h, w3_ref[...], preferred_element_type=jnp.float32) + b3_ref[...])


def _full(shape):
    return pl.BlockSpec(shape, lambda i: tuple(0 for _ in shape))


_mlp_call = pl.pallas_call(
    _mlp_body,
    grid=(B // _B_BLK,),
    in_specs=[
        pl.BlockSpec((S, _B_BLK), lambda i: (0, i)),
        pl.BlockSpec((S, _B_BLK), lambda i: (0, i)),
        pl.BlockSpec((S, _B_BLK), lambda i: (0, i)),
        pl.BlockSpec((OTHER, _B_BLK), lambda i: (0, i)),
        _full((S, 256)),
        _full((S, 256)),
        _full((S, 256)),
        _full((OTHER, 256)),
        _full((1, 256)),
        _full((256, 128)),
        _full((1, 128)),
        _full((128, 128)),
        _full((1, 128)),
    ],
    out_specs=pl.BlockSpec((_B_BLK, 128), lambda i: (i, 0)),
    out_shape=jax.ShapeDtypeStruct((B, 128), jnp.float32),
)


def kernel(embedding, gene_with_go_idx, gene_with_go_value,
           gene_without_go_value, other_info, W1, b1, W2, b2, W3, b3):
    sums = _rowsum_call(embedding.T)
    idx_flat = gene_with_go_idx.T.reshape(BS).astype(jnp.int32)
    gathered = _sc_gather_call()(sums, idx_flat)
    g_t = gathered.reshape(S, B)
    w1a = W1[0:S]
    w1b = W1[S:2 * S]
    w1c = W1[2 * S:3 * S]
    w1d = W1[3 * S:]
    return _mlp_call(
        g_t, gene_with_go_value.T, gene_without_go_value.T, other_info.T,
        w1a, w1b, w1c, w1d, b1.reshape(1, 256),
        W2, b2.reshape(1, 128), W3, b3.reshape(1, 128))


# speedup vs baseline: 6.3528x; 1.2534x over previous
import functools

import jax
import jax.numpy as jnp
import numpy as np
from jax import lax
from jax.experimental import pallas as pl
from jax.experimental.pallas import tpu as pltpu
from jax.experimental.pallas import tpu_sc as plsc

B, S, D, V = 4096, 200, 64, 1000000
BS = B * S
OTHER = 64


_COLS_BLK = 32768
_N_BLKS = -(-V // _COLS_BLK)
_VPAD = _N_BLKS * _COLS_BLK


def _rowsum_body(emb_ref, out_ref):
    out_ref[...] = jnp.sum(emb_ref[...], axis=0)


_rowsum_call = pl.pallas_call(
    _rowsum_body,
    grid=(_N_BLKS,),
    in_specs=[pl.BlockSpec((D, _COLS_BLK), lambda i: (0, i))],
    out_specs=pl.BlockSpec((_COLS_BLK,), lambda i: (i,)),
    out_shape=jax.ShapeDtypeStruct((_VPAD,), jnp.float32),
)


_NC, _NS = 2, 16
_NW = _NC * _NS
_N_PER = BS // _NW


def _sc_gather_body(sums_hbm, idx_hbm, out_hbm, idx_v, vals_v, sem):
    wid = lax.axis_index("s") * _NC + lax.axis_index("c")
    base = wid * _N_PER
    pltpu.sync_copy(idx_hbm.at[pl.ds(base, _N_PER)], idx_v)
    pltpu.async_copy(sums_hbm.at[idx_v], vals_v, sem).wait()
    pltpu.sync_copy(vals_v, out_hbm.at[pl.ds(base, _N_PER)])


@functools.cache
def _sc_gather_call():
    return pl.kernel(
        _sc_gather_body,
        out_type=jax.ShapeDtypeStruct((BS,), jnp.float32),
        mesh=plsc.VectorSubcoreMesh(
            core_axis_name="c", subcore_axis_name="s", num_cores=_NC,
            num_subcores=_NS),
        scratch_types=[
            pltpu.VMEM((_N_PER,), jnp.int32),
            pltpu.VMEM((_N_PER,), jnp.float32),
            pltpu.SemaphoreType.DMA,
        ],
    )


_B_BLK = 512


def _tdot(a_t, w):
    return lax.dot_general(a_t, w, (((0,), (0,)), ((), ())),
                           preferred_element_type=jnp.float32)


def _mlp_body(g_ref, gwv_ref, gwov_ref, oth_ref, w1a_ref, w1b_ref, w1c_ref,
              w1d_ref, b1_ref, w2_ref, b2_ref, w3_ref, b3_ref, out_ref):
    gwv_t = gwv_ref[...]
    emb_t = g_ref[...] * gwv_t * (1.0 / D)
    h = (_tdot(emb_t, w1a_ref[...]) + _tdot(gwv_t, w1b_ref[...])
         + _tdot(gwov_ref[...], w1c_ref[...]) + _tdot(oth_ref[...], w1d_ref[...])
         + b1_ref[...])
    h = jnp.maximum(h, 0.0)
    h = jnp.maximum(
        jnp.dot(h, w2_ref[...], preferred_element_type=jnp.float32) + b2_ref[...],
        0.0)
    out_ref[...] = (
        jnp.dot(h, w3_ref[...], preferred_element_type=jnp.float32) + b3_ref[...])


def _full(shape):
    return pl.BlockSpec(shape, lambda i: tuple(0 for _ in shape))


_mlp_call = pl.pallas_call(
    _mlp_body,
    grid=(B // _B_BLK,),
    in_specs=[
        pl.BlockSpec((S, _B_BLK), lambda i: (0, i)),
        pl.BlockSpec((S, _B_BLK), lambda i: (0, i)),
        pl.BlockSpec((S, _B_BLK), lambda i: (0, i)),
        pl.BlockSpec((OTHER, _B_BLK), lambda i: (0, i)),
        _full((S, 256)),
        _full((S, 256)),
        _full((S, 256)),
        _full((OTHER, 256)),
        _full((1, 256)),
        _full((256, 128)),
        _full((1, 128)),
        _full((128, 128)),
        _full((1, 128)),
    ],
    out_specs=pl.BlockSpec((_B_BLK, 128), lambda i: (i, 0)),
    out_shape=jax.ShapeDtypeStruct((B, 128), jnp.float32),
)


def kernel(embedding, gene_with_go_idx, gene_with_go_value,
           gene_without_go_value, other_info, W1, b1, W2, b2, W3, b3):
    sums = _rowsum_call(embedding.T)
    idx_flat = gene_with_go_idx.T.reshape(BS).astype(jnp.int32)
    gathered = _sc_gather_call()(sums, idx_flat)
    g_t = gathered.reshape(S, B)
    w1a = W1[0:S]
    w1b = W1[S:2 * S]
    w1c = W1[2 * S:3 * S]
    w1d = W1[3 * S:]
    return _mlp_call(
        g_t, gene_with_go_value.T, gene_without_go_value.T, other_info.T,
        w1a, w1b, w1c, w1d, b1.reshape(1, 256),
        W2, b2.reshape(1, 128), W3, b3.reshape(1, 128))
